# baseline (speedup 1.0000x reference)
import functools

import jax
import jax.numpy as jnp
from jax import lax
from jax.experimental import pallas as pl
from jax.experimental.pallas import tpu as pltpu
from jax.experimental.pallas import tpu_sc as plsc

_N = 10000
_E = 320000
_EP = 327680
_NC = 2
_NS = 16
_RPW = 80
_EROWS = _EP // 128
_EPW = _EP // (_NC * _NS)
_NPAD = 10240
_TRASH = _N
_HROWS = _NPAD // 16

@functools.lru_cache(maxsize=None)
def _sc_mesh():
    return plsc.VectorSubcoreMesh(core_axis_name="c", subcore_axis_name="s",
                                  num_cores=_NC, num_subcores=_NS)


def _deg_body(dst_hbm, out_hbm, hist, dstbuf):
    c = lax.axis_index("c")
    s = lax.axis_index("s")
    wid = c * _NS + s
    pltpu.sync_copy(dst_hbm.at[pl.ds(wid * _RPW, _RPW)], dstbuf)

    zero16 = jnp.zeros((16,), jnp.float32)

    def _zrow(r, carry):
        hist[pl.ds(r * 16, 16)] = zero16
        return carry

    lax.fori_loop(0, _HROWS, _zrow, 0)

    ones16 = jnp.ones((16,), jnp.float32)

    def _hrow(r, carry):
        for q in range(8):
            n = dstbuf[r, pl.ds(q * 16, 16)]
            plsc.addupdate_scatter(hist, [n], ones16)
        return carry

    lax.fori_loop(0, _RPW, _hrow, 0)

    pltpu.sync_copy(hist, out_hbm.at[c, s])


@functools.lru_cache(maxsize=None)
def _deg_call():
    return pl.kernel(
        _deg_body,
        out_type=jax.ShapeDtypeStruct((_NC, _NS, _NPAD), jnp.float32),
        mesh=_sc_mesh(),
        compiler_params=pltpu.CompilerParams(needs_layout_passes=False),
        scratch_types=[
            pltpu.VMEM((_NPAD,), jnp.float32),
            pltpu.VMEM((_RPW, 128), jnp.int32),
        ],
    )


_CH = 64
_CPW = _EPW // _CH
_HB = 80


def _prop_body(g_hbm, src_hbm, dst_hbm, out_hbm, srcbuf, dstbuf, rows,
               accum, sem0, sem1):
    c = lax.axis_index("c")
    s = lax.axis_index("s")
    wid = c * _NS + s
    base = wid * _CPW

    zero16 = jnp.zeros((16,), jnp.float32)

    with jax.named_scope("prop_zero"):
        def _zrow(r, carry):
            for b in range(2):
                for q in range(8):
                    rows[b, r, pl.ds(q * 16, 16)] = zero16
            return carry

        lax.fori_loop(0, _CH, _zrow, 0)
        for j in range(10):
            pltpu.sync_copy(rows.at[0],
                            accum.at[pl.ds(s * 640 + j * _CH, _CH)])
        plsc.subcore_barrier()

    with jax.named_scope("prop_acc"):
        sems = (sem0, sem1)
        for h in range(_CPW // _HB):
            hb = base + h * _HB
            pltpu.sync_copy(src_hbm.at[pl.ds(hb, _HB)], srcbuf)
            pltpu.sync_copy(dst_hbm.at[pl.ds(hb, _HB)], dstbuf)
            for b in range(2):
                pltpu.async_copy(g_hbm.at[srcbuf.at[b]], rows.at[b], sems[b])

            def _pair(p, carry):
                for b in range(2):
                    k = 2 * p + b
                    pltpu.make_async_copy(g_hbm.at[srcbuf.at[k]], rows.at[b],
                                          sems[b]).wait()
                    pltpu.sync_copy(rows.at[b], accum.at[dstbuf.at[k]],
                                    add=True)

                    @pl.when(p < _HB // 2 - 1)
                    def _():
                        pltpu.async_copy(g_hbm.at[srcbuf.at[k + 2]],
                                         rows.at[b], sems[b])
                return carry

            lax.fori_loop(0, _HB // 2, _pair, 0)
        plsc.subcore_barrier()

    with jax.named_scope("prop_dump"):
        for j in range(10):
            r0 = s * 640 + j * _CH
            pltpu.sync_copy(accum.at[pl.ds(r0, _CH)],
                            out_hbm.at[c, pl.ds(r0, _CH)])


@functools.lru_cache(maxsize=None)
def _prop_call():
    return pl.kernel(
        _prop_body,
        out_type=jax.ShapeDtypeStruct((_NC, _NPAD, 128), jnp.float32),
        mesh=_sc_mesh(),
        compiler_params=pltpu.CompilerParams(needs_layout_passes=False),
        scratch_types=[
            pltpu.VMEM((_HB, _CH), jnp.int32),
            pltpu.VMEM((_HB, _CH), jnp.int32),
            pltpu.VMEM((2, _CH, 128), jnp.float32),
            pltpu.VMEM_SHARED((_NPAD, 128), jnp.float32),
            pltpu.SemaphoreType.DMA,
            pltpu.SemaphoreType.DMA,
        ],
    )


def _tc_prep_body(deg_ref, x_ref, g_ref):
    dinv = lax.rsqrt(deg_ref[...] + 1.0)
    g_ref[...] = dinv * x_ref[...]


def _tc_layer1_body(deg_ref, s_ref, g1_ref, w1_ref, b1_ref, g2_ref):
    dinv = lax.rsqrt(deg_ref[...] + 1.0)
    agg = dinv * (s_ref[0] + s_ref[1] + g1_ref[...])
    h = jnp.dot(agg, w1_ref[...], preferred_element_type=jnp.float32)
    h = jnp.maximum(h + b1_ref[...], 0.0)
    g2_ref[...] = dinv * h


def _tc_layer2_body(deg_ref, sa_ref, sb_ref, g2_ref, w2_ref, b2_ref, w3_ref,
                    g3_ref):
    dinv = lax.rsqrt(deg_ref[...] + 1.0)
    scat = jnp.concatenate([sa_ref[0] + sa_ref[1], sb_ref[0] + sb_ref[1]],
                           axis=1)
    agg = dinv * (scat + g2_ref[...])
    h = jnp.dot(agg, w2_ref[...], preferred_element_type=jnp.float32)
    h = jnp.maximum(h + b2_ref[...], 0.0)
    t = jnp.dot(h, w3_ref[...], preferred_element_type=jnp.float32)
    g3_ref[...] = dinv * t


def _tc_post_body(deg_ref, s_ref, g3_ref, b3_ref, out_ref):
    dinv = lax.rsqrt(deg_ref[...] + 1.0)
    out_ref[...] = dinv * (s_ref[0] + s_ref[1] + g3_ref[...]) + b3_ref[...]


_tc_prep = pl.pallas_call(
    _tc_prep_body,
    out_shape=jax.ShapeDtypeStruct((_NPAD, 128), jnp.float32),
)

_tc_layer1 = pl.pallas_call(
    _tc_layer1_body,
    out_shape=jax.ShapeDtypeStruct((_NPAD, 256), jnp.float32),
)

_tc_layer2 = pl.pallas_call(
    _tc_layer2_body,
    out_shape=jax.ShapeDtypeStruct((_NPAD, 128), jnp.float32),
)

_tc_post = pl.pallas_call(
    _tc_post_body,
    out_shape=jax.ShapeDtypeStruct((_NPAD, 128), jnp.float32),
)


def kernel(x, edge_index, W1, b1, W2, b2, W3, b3):
    src = edge_index[0].astype(jnp.int32)
    dst = edge_index[1].astype(jnp.int32)
    src_flat = jnp.concatenate([src, jnp.zeros((_EP - _E,), jnp.int32)])
    dst_flat = jnp.concatenate(
        [dst, jnp.full((_EP - _E,), _TRASH, jnp.int32)])
    dst2 = dst_flat.reshape(_EROWS, 128)
    sc1_off = jnp.where(jnp.arange(_EP, dtype=jnp.int32) >= _EP // 2,
                        jnp.int32(_NPAD), jnp.int32(0))
    src64 = (src_flat + sc1_off).reshape(_EP // _CH, _CH)
    dst64 = dst_flat.reshape(_EP // _CH, _CH)

    deg2 = _deg_call()(dst2)
    deg = deg2.sum(axis=(0, 1)).reshape(_NPAD, 1)

    xp = jnp.concatenate(
        [x, jnp.zeros((_NPAD - _N, x.shape[1]), x.dtype)])
    g1 = _tc_prep(deg, xp)
    s1 = _prop_call()(jnp.concatenate([g1, g1]), src64, dst64)
    g2 = _tc_layer1(deg, s1, g1, W1, b1.reshape(1, -1))
    g2a = g2[:, :128]
    g2b = g2[:, 128:]
    s2a = _prop_call()(jnp.concatenate([g2a, g2a]), src64, dst64)
    s2b = _prop_call()(jnp.concatenate([g2b, g2b]), src64, dst64)
    g3 = _tc_layer2(deg, s2a, s2b, g2, W2, b2.reshape(1, -1), W3)
    s3 = _prop_call()(jnp.concatenate([g3, g3]), src64, dst64)
    out = _tc_post(deg, s3, g3, b3.reshape(1, -1))
    return out[:_N]


# speedup vs baseline: 1.3124x; 1.3124x over previous
import functools

import jax
import jax.numpy as jnp
from jax import lax
from jax.experimental import pallas as pl
from jax.experimental.pallas import tpu as pltpu
from jax.experimental.pallas import tpu_sc as plsc

_N = 10000
_E = 320000
_EP = 327680
_NC = 2
_NS = 16
_RPW = 80
_EROWS = _EP // 128
_EPW = _EP // (_NC * _NS)
_NPAD = 10240
_TRASH = _N
_HROWS = _NPAD // 16

@functools.lru_cache(maxsize=None)
def _sc_mesh():
    return plsc.VectorSubcoreMesh(core_axis_name="c", subcore_axis_name="s",
                                  num_cores=_NC, num_subcores=_NS)


def _deg_body(dst_hbm, out_hbm, hist, dstbuf):
    c = lax.axis_index("c")
    s = lax.axis_index("s")
    wid = c * _NS + s
    pltpu.sync_copy(dst_hbm.at[pl.ds(wid * _RPW, _RPW)], dstbuf)

    zero16 = jnp.zeros((16,), jnp.float32)

    def _zrow(r, carry):
        hist[pl.ds(r * 16, 16)] = zero16
        return carry

    lax.fori_loop(0, _HROWS, _zrow, 0)

    ones16 = jnp.ones((16,), jnp.float32)

    def _hrow(r, carry):
        for q in range(8):
            n = dstbuf[r, pl.ds(q * 16, 16)]
            plsc.addupdate_scatter(hist, [n], ones16)
        return carry

    lax.fori_loop(0, _RPW, _hrow, 0)

    pltpu.sync_copy(hist, out_hbm.at[c, s])


@functools.lru_cache(maxsize=None)
def _deg_call():
    return pl.kernel(
        _deg_body,
        out_type=jax.ShapeDtypeStruct((_NC, _NS, _NPAD), jnp.float32),
        mesh=_sc_mesh(),
        compiler_params=pltpu.CompilerParams(needs_layout_passes=False),
        scratch_types=[
            pltpu.VMEM((_NPAD,), jnp.float32),
            pltpu.VMEM((_RPW, 128), jnp.int32),
        ],
    )


_CH = 64
_CPW = _EPW // _CH
_HB = 80
_C0 = 240
_C1 = 80
_STAGES0 = (80, 80, 80)
_STAGES1 = (80,)


def _prop_body(g_hbm, src_hbm, dst_hbm, out_hbm, srcbuf, dstbuf, rows,
               accum, sem0, sem1):
    c = lax.axis_index("c")
    s = lax.axis_index("s")

    zero16 = jnp.zeros((16,), jnp.float32)

    with jax.named_scope("prop_zero"):
        def _zrow(r, carry):
            for b in range(2):
                for q in range(8):
                    rows[b, r, pl.ds(q * 16, 16)] = zero16
            return carry

        lax.fori_loop(0, _CH, _zrow, 0)
        for j in range(10):
            pltpu.sync_copy(rows.at[0],
                            accum.at[pl.ds(s * 640 + j * _CH, _CH)])
        plsc.subcore_barrier()

    with jax.named_scope("prop_acc"):
        sems = (sem0, sem1)

        def _run(cbase, stages):
            off = 0
            for sz in stages:
                hb = cbase + off
                pltpu.sync_copy(src_hbm.at[pl.ds(hb, sz)],
                                srcbuf.at[pl.ds(0, sz)])
                pltpu.sync_copy(dst_hbm.at[pl.ds(hb, sz)],
                                dstbuf.at[pl.ds(0, sz)])
                for b in range(2):
                    pltpu.async_copy(g_hbm.at[srcbuf.at[b]], rows.at[b],
                                     sems[b])

                def _pair(p, carry):
                    for b in range(2):
                        k = 2 * p + b
                        pltpu.make_async_copy(g_hbm.at[srcbuf.at[k]],
                                              rows.at[b], sems[b]).wait()
                        pltpu.sync_copy(rows.at[b], accum.at[dstbuf.at[k]],
                                        add=True)

                        @pl.when(p < sz // 2 - 1)
                        def _():
                            pltpu.async_copy(g_hbm.at[srcbuf.at[k + 2]],
                                             rows.at[b], sems[b])
                    return carry

                lax.fori_loop(0, sz // 2, _pair, 0)
                off += sz

        @pl.when(c == 0)
        def _():
            _run(s * _C0, _STAGES0)

        @pl.when(c == 1)
        def _():
            _run(_NS * _C0 + s * _C1, _STAGES1)

        plsc.subcore_barrier()

    with jax.named_scope("prop_dump"):
        for j in range(10):
            r0 = s * 640 + j * _CH
            pltpu.sync_copy(accum.at[pl.ds(r0, _CH)],
                            out_hbm.at[c, pl.ds(r0, _CH)])


@functools.lru_cache(maxsize=None)
def _prop_call():
    return pl.kernel(
        _prop_body,
        out_type=jax.ShapeDtypeStruct((_NC, _NPAD, 128), jnp.float32),
        mesh=_sc_mesh(),
        compiler_params=pltpu.CompilerParams(needs_layout_passes=False),
        scratch_types=[
            pltpu.VMEM((_HB, _CH), jnp.int32),
            pltpu.VMEM((_HB, _CH), jnp.int32),
            pltpu.VMEM((2, _CH, 128), jnp.float32),
            pltpu.VMEM_SHARED((_NPAD, 128), jnp.float32),
            pltpu.SemaphoreType.DMA,
            pltpu.SemaphoreType.DMA,
        ],
    )


def _tc_prep_body(deg_ref, x_ref, g_ref):
    dinv = lax.rsqrt(deg_ref[...] + 1.0)
    g_ref[...] = dinv * x_ref[...]


def _tc_layer1_body(deg_ref, s_ref, g1_ref, w1_ref, b1_ref, g2_ref):
    dinv = lax.rsqrt(deg_ref[...] + 1.0)
    agg = dinv * (s_ref[0] + s_ref[1] + g1_ref[...])
    h = jnp.dot(agg, w1_ref[...], preferred_element_type=jnp.float32)
    h = jnp.maximum(h + b1_ref[...], 0.0)
    g2_ref[...] = dinv * h


def _tc_layer2_body(deg_ref, sa_ref, sb_ref, g2_ref, w2_ref, b2_ref, w3_ref,
                    g3_ref):
    dinv = lax.rsqrt(deg_ref[...] + 1.0)
    scat = jnp.concatenate([sa_ref[0] + sa_ref[1], sb_ref[0] + sb_ref[1]],
                           axis=1)
    agg = dinv * (scat + g2_ref[...])
    h = jnp.dot(agg, w2_ref[...], preferred_element_type=jnp.float32)
    h = jnp.maximum(h + b2_ref[...], 0.0)
    t = jnp.dot(h, w3_ref[...], preferred_element_type=jnp.float32)
    g3_ref[...] = dinv * t


def _tc_post_body(deg_ref, s_ref, g3_ref, b3_ref, out_ref):
    dinv = lax.rsqrt(deg_ref[...] + 1.0)
    out_ref[...] = dinv * (s_ref[0] + s_ref[1] + g3_ref[...]) + b3_ref[...]


_tc_prep = pl.pallas_call(
    _tc_prep_body,
    out_shape=jax.ShapeDtypeStruct((_NPAD, 128), jnp.float32),
)

_tc_layer1 = pl.pallas_call(
    _tc_layer1_body,
    out_shape=jax.ShapeDtypeStruct((_NPAD, 256), jnp.float32),
)

_tc_layer2 = pl.pallas_call(
    _tc_layer2_body,
    out_shape=jax.ShapeDtypeStruct((_NPAD, 128), jnp.float32),
)

_tc_post = pl.pallas_call(
    _tc_post_body,
    out_shape=jax.ShapeDtypeStruct((_NPAD, 128), jnp.float32),
)


def kernel(x, edge_index, W1, b1, W2, b2, W3, b3):
    src = edge_index[0].astype(jnp.int32)
    dst = edge_index[1].astype(jnp.int32)
    src_flat = jnp.concatenate([src, jnp.zeros((_EP - _E,), jnp.int32)])
    dst_flat = jnp.concatenate(
        [dst, jnp.full((_EP - _E,), _TRASH, jnp.int32)])
    dst2 = dst_flat.reshape(_EROWS, 128)
    src64 = src_flat.reshape(_EP // _CH, _CH)
    dst64 = dst_flat.reshape(_EP // _CH, _CH)

    deg2 = _deg_call()(dst2)
    deg = deg2.sum(axis=(0, 1)).reshape(_NPAD, 1)

    xp = jnp.concatenate(
        [x, jnp.zeros((_NPAD - _N, x.shape[1]), x.dtype)])
    g1 = _tc_prep(deg, xp)
    s1 = _prop_call()(g1, src64, dst64)
    g2 = _tc_layer1(deg, s1, g1, W1, b1.reshape(1, -1))
    g2a = g2[:, :128]
    g2b = g2[:, 128:]
    s2a = _prop_call()(g2a, src64, dst64)
    s2b = _prop_call()(g2b, src64, dst64)
    g3 = _tc_layer2(deg, s2a, s2b, g2, W2, b2.reshape(1, -1), W3)
    s3 = _prop_call()(g3, src64, dst64)
    out = _tc_post(deg, s3, g3, b3.reshape(1, -1))
    return out[:_N]
